# baseline (speedup 1.0000x reference)
import functools

import jax
import jax.numpy as jnp
from jax import lax
from jax.experimental import pallas as pl
from jax.experimental.pallas import tpu as pltpu
from jax.experimental.pallas import tpu_sc as plsc

def _dotT(a, w):
    return lax.dot_general(
        a, w, (((1,), (1,)), ((), ())), preferred_element_type=jnp.float32
    )


V = 16000
EMB = 512
H = 768
NH4 = 4 * H
B = 8
S = 256
TMAX = 16
NVC = 5
CV = V // NVC


def _embed_gather(table, idx):
    info = plsc.get_sparse_core_info()
    nc, ns = info.num_cores, info.num_subcores
    nw = nc * ns
    n = idx.shape[0]
    d = table.shape[1]
    b_per_w = n // nw
    mesh = plsc.VectorSubcoreMesh(core_axis_name="c", subcore_axis_name="s")

    @functools.partial(
        pl.kernel,
        mesh=mesh,
        out_type=jax.ShapeDtypeStruct((n, d), jnp.float32),
        scratch_types=[
            pltpu.VMEM((b_per_w,), jnp.int32),
            pltpu.VMEM((b_per_w, d), jnp.float32),
            pltpu.SemaphoreType.DMA,
        ],
    )
    def k(table_hbm, idx_hbm, out_hbm, idx_v, rows_v, sem):
        wid = lax.axis_index("s") * nc + lax.axis_index("c")
        base = wid * b_per_w
        pltpu.sync_copy(idx_hbm.at[pl.ds(base, b_per_w)], idx_v)
        pltpu.async_copy(table_hbm.at[idx_v], rows_v, sem).wait()
        pltpu.sync_copy(rows_v, out_hbm.at[pl.ds(base, b_per_w)])

    return k(table, idx)


def _mm_body(x_ref, w_ref, b_ref, o_ref):
    o_ref[...] = (
        jnp.dot(x_ref[...], w_ref[...], preferred_element_type=jnp.float32)
        + b_ref[...]
    )


def _matmul_bias(x, wT, b):
    m, k = x.shape
    n = wT.shape[1]
    bn = 768
    return pl.pallas_call(
        _mm_body,
        grid=(n // bn,),
        in_specs=[
            pl.BlockSpec((m, k), lambda j: (0, 0)),
            pl.BlockSpec((k, bn), lambda j: (0, j)),
            pl.BlockSpec((1, bn), lambda j: (0, j)),
        ],
        out_specs=pl.BlockSpec((m, bn), lambda j: (0, j)),
        out_shape=jax.ShapeDtypeStruct((m, n), jnp.float32),
        compiler_params=pltpu.CompilerParams(
            dimension_semantics=("arbitrary",),
        ),
    )(x, wT, b.reshape(1, n))


def _lstm_body(g_ref, whh_ref, hseq_ref, cfin_ref, h_sc, c_sc):
    t = pl.program_id(0)

    @pl.when(t == 0)
    def _():
        h_sc[...] = jnp.zeros_like(h_sc)
        c_sc[...] = jnp.zeros_like(c_sc)

    g = g_ref[0] + jnp.dot(
        h_sc[...], whh_ref[...], preferred_element_type=jnp.float32
    )
    ig = jax.nn.sigmoid(g[:, :H])
    fg = jax.nn.sigmoid(g[:, H : 2 * H])
    gg = jnp.tanh(g[:, 2 * H : 3 * H])
    og = jax.nn.sigmoid(g[:, 3 * H :])
    c2 = fg * c_sc[...] + ig * gg
    h2 = og * jnp.tanh(c2)
    h_sc[...] = h2
    c_sc[...] = c2
    hseq_ref[0] = h2

    @pl.when(t == S - 1)
    def _():
        cfin_ref[...] = c2


def _enc_body(xa_ref, xb_ref, wih0, whh0, wih1, whh1, b0, b1,
              h0fin, h1fin, c1fin, h0_sc, c0_sc, h1_sc, c1_sc, g0x, g1x):
    t = pl.program_id(0)

    def _cell(g, c_prev):
        ig = jax.nn.sigmoid(g[:, :H])
        fg = jax.nn.sigmoid(g[:, H : 2 * H])
        gg = jnp.tanh(g[:, 2 * H : 3 * H])
        og = jax.nn.sigmoid(g[:, 3 * H :])
        c2 = fg * c_prev + ig * gg
        return og * jnp.tanh(c2), c2

    @pl.when(t == 0)
    def _():
        h0_sc[...] = jnp.zeros_like(h0_sc)
        c0_sc[...] = jnp.zeros_like(c0_sc)
        h1_sc[...] = jnp.zeros_like(h1_sc)
        c1_sc[...] = jnp.zeros_like(c1_sc)
        g0x[...] = _dotT(xb_ref[0], wih0[...]) + b0[...]

    @pl.when(t >= 1)
    def _():
        g = g1x[...] + _dotT(h1_sc[...], whh1[...])
        h2, c2 = _cell(g, c1_sc[...])
        h1_sc[...] = h2
        c1_sc[...] = c2

        @pl.when(t == S)
        def _():
            h1fin[...] = h2
            c1fin[...] = c2

    @pl.when(t < S)
    def _():
        g = g0x[...] + _dotT(h0_sc[...], whh0[...])
        h2, c2 = _cell(g, c0_sc[...])
        h0_sc[...] = h2
        c0_sc[...] = c2
        g1x[...] = _dotT(h2, wih1[...]) + b1[...]
        g0x[...] = _dotT(xa_ref[0], wih0[...]) + b0[...]

        @pl.when(t == S - 1)
        def _():
            h0fin[...] = h2


def _encoder_fused(xseq, wih0T, whh0T, wih1T, whh1T, b0, b1):
    fixed = lambda t: (0, 0)
    return pl.pallas_call(
        _enc_body,
        grid=(S + 1,),
        in_specs=[
            pl.BlockSpec((1, B, EMB), lambda t: (jnp.minimum(t + 1, S - 1), 0, 0)),
            pl.BlockSpec((1, B, EMB), lambda t: (0, 0, 0)),
            pl.BlockSpec((NH4, EMB), fixed),
            pl.BlockSpec((NH4, H), fixed),
            pl.BlockSpec((NH4, H), fixed),
            pl.BlockSpec((NH4, H), fixed),
            pl.BlockSpec((1, NH4), fixed),
            pl.BlockSpec((1, NH4), fixed),
        ],
        out_specs=[
            pl.BlockSpec((B, H), fixed),
            pl.BlockSpec((B, H), fixed),
            pl.BlockSpec((B, H), fixed),
        ],
        out_shape=[
            jax.ShapeDtypeStruct((B, H), jnp.float32),
            jax.ShapeDtypeStruct((B, H), jnp.float32),
            jax.ShapeDtypeStruct((B, H), jnp.float32),
        ],
        scratch_shapes=[
            pltpu.VMEM((B, H), jnp.float32),
            pltpu.VMEM((B, H), jnp.float32),
            pltpu.VMEM((B, H), jnp.float32),
            pltpu.VMEM((B, H), jnp.float32),
            pltpu.VMEM((B, NH4), jnp.float32),
            pltpu.VMEM((B, NH4), jnp.float32),
        ],
        compiler_params=pltpu.CompilerParams(
            dimension_semantics=("arbitrary",),
            vmem_limit_bytes=60 * 1024 * 1024,
        ),
    )(xseq, xseq, wih0T, whh0T, wih1T, whh1T,
      b0.reshape(1, NH4), b1.reshape(1, NH4))


def _mm8_body(x_ref, w_ref, b_ref, o_ref):
    o_ref[0] = (
        jnp.dot(x_ref[0], w_ref[...], preferred_element_type=jnp.float32)
        + b_ref[...]
    )


def _matmul_rows(xseq, wT, b):
    k = xseq.shape[2]
    return pl.pallas_call(
        _mm8_body,
        grid=(S,),
        in_specs=[
            pl.BlockSpec((1, B, k), lambda t: (t, 0, 0)),
            pl.BlockSpec((k, NH4), lambda t: (0, 0)),
            pl.BlockSpec((1, NH4), lambda t: (0, 0)),
        ],
        out_specs=pl.BlockSpec((1, B, NH4), lambda t: (t, 0, 0)),
        out_shape=jax.ShapeDtypeStruct((S, B, NH4), jnp.float32),
        compiler_params=pltpu.CompilerParams(
            dimension_semantics=("arbitrary",),
        ),
    )(xseq, wT, b.reshape(1, NH4))


def _lstm_scan(g, whhT):
    return pl.pallas_call(
        _lstm_body,
        grid=(S,),
        in_specs=[
            pl.BlockSpec((1, B, NH4), lambda t: (t, 0, 0)),
            pl.BlockSpec((H, NH4), lambda t: (0, 0)),
        ],
        out_specs=[
            pl.BlockSpec((1, B, H), lambda t: (t, 0, 0)),
            pl.BlockSpec((B, H), lambda t: (0, 0)),
        ],
        out_shape=[
            jax.ShapeDtypeStruct((S, B, H), jnp.float32),
            jax.ShapeDtypeStruct((B, H), jnp.float32),
        ],
        scratch_shapes=[
            pltpu.VMEM((B, H), jnp.float32),
            pltpu.VMEM((B, H), jnp.float32),
        ],
        compiler_params=pltpu.CompilerParams(
            dimension_semantics=("arbitrary",),
        ),
    )(g, whhT)


def _dec_body(
    eh0, eh1, ec1, wih, whh, bih, bhh, wq, wk, wv, wo, bq, bk, bv, bo,
    wlin, blin, emb_tab,
    douts_ref, res_ref,
    h_sc, c_sc, attn_sc, emb_sc, ke0, ke1, ve0, ve1, cv_sc, ct_sc,
    g_sc, tokv, toks, esem, psem,
):
    t = pl.program_id(0)
    v = pl.program_id(1)

    def _fetch_emb_and_project():
        for i in range(B):
            pltpu.make_async_copy(
                emb_tab.at[pl.ds(toks[i, 0], 1), :],
                emb_sc.at[pl.ds(i, 1), :],
                esem,
            ).start()
        for i in range(B):
            pltpu.make_async_copy(
                emb_tab.at[pl.ds(toks[i, 0], 1), :],
                emb_sc.at[pl.ds(i, 1), :],
                esem,
            ).wait()
        g_sc[...] = _dotT(emb_sc[...], wih[...])

    @pl.when(v == 0)
    def _step_head():
        @pl.when(t == 0)
        def _():
            h_sc[...] = eh1[...]
            c_sc[...] = ec1[...]
            ke0[...] = _dotT(eh0[...], wk[...]) + bk[...]
            ke1[...] = _dotT(eh1[...], wk[...]) + bk[...]
            ve0[...] = _dotT(eh0[...], wv[...]) + bv[...]
            ve1[...] = _dotT(eh1[...], wv[...]) + bv[...]
            for i in range(B):
                toks[i, 0] = 1
            _fetch_emb_and_project()

        cv_sc[...] = jnp.full_like(cv_sc, -jnp.inf)

        g = g_sc[...] + _dotT(h_sc[...], whh[...]) + bih[...] + bhh[...]
        ig = jax.nn.sigmoid(g[:, :H])
        fg = jax.nn.sigmoid(g[:, H : 2 * H])
        gg = jnp.tanh(g[:, 2 * H : 3 * H])
        og = jax.nn.sigmoid(g[:, 3 * H :])
        c2 = fg * c_sc[...] + ig * gg
        h2 = og * jnp.tanh(c2)
        h_sc[...] = h2
        c_sc[...] = c2

        q = _dotT(h2, wq[...]) + bq[...]
        kd = _dotT(h2, wk[...]) + bk[...]
        vd = _dotT(h2, wv[...]) + bv[...]
        kall = jnp.concatenate([ke0[...], ke1[...], kd], axis=0)
        vall = jnp.concatenate([ve0[...], ve1[...], vd], axis=0)
        s24 = lax.dot_general(
            q, kall, (((1,), (1,)), ((), ())),
            preferred_element_type=jnp.float32,
        )
        lane24 = lax.broadcasted_iota(jnp.int32, (B, 3 * B), 1)
        row24 = lax.broadcasted_iota(jnp.int32, (B, 3 * B), 0)
        sq = jnp.sqrt(jnp.float32(H))
        sel = [lane24 == (j * B + row24) for j in range(3)]
        s0 = jnp.sum(jnp.where(sel[0], s24, 0.0), 1, keepdims=True) / sq
        s1 = jnp.sum(jnp.where(sel[1], s24, 0.0), 1, keepdims=True) / sq
        s2 = jnp.sum(jnp.where(sel[2], s24, 0.0), 1, keepdims=True) / sq
        mx = jnp.maximum(jnp.maximum(s0, s1), s2)
        e0 = jnp.exp(s0 - mx)
        e1 = jnp.exp(s1 - mx)
        e2 = jnp.exp(s2 - mx)
        den = e0 + e1 + e2
        wfull = (
            jnp.where(sel[0], e0 / den, 0.0)
            + jnp.where(sel[1], e1 / den, 0.0)
            + jnp.where(sel[2], e2 / den, 0.0)
        )
        at = jnp.dot(wfull, vall, preferred_element_type=jnp.float32)
        attn_sc[...] = _dotT(at, wo[...]) + bo[...]

    logits = _dotT(attn_sc[...], wlin[...]) + blin[...]
    douts_ref[0] = logits
    lane = lax.broadcasted_iota(jnp.int32, (B, CV), 1)
    mx = jnp.max(logits, axis=1, keepdims=True)
    l0 = jnp.min(jnp.where(logits == mx, lane, CV), axis=1, keepdims=True)
    lane128 = lax.broadcasted_iota(jnp.int32, (B, 128), 1)
    upd = lane128 == v
    cv_sc[...] = jnp.where(upd, mx, cv_sc[...])
    ct_sc[...] = jnp.where(upd, l0 + v * CV, ct_sc[...])

    @pl.when(v == NVC - 1)
    def _step_tail():
        mz = jnp.max(cv_sc[...], axis=1, keepdims=True)
        lz = jnp.min(
            jnp.where(cv_sc[...] == mz, lane128, 128), axis=1, keepdims=True
        )
        tokv[...] = jnp.sum(
            jnp.where(lane128 == lz, ct_sc[...], 0), axis=1, keepdims=True
        )
        cp = pltpu.make_async_copy(tokv, toks, psem)
        cp.start()
        cp.wait()

        @pl.when(t == 0)
        def _():
            for i in range(B):
                res_ref[i, 0] = 1

        for i in range(B):
            res_ref[i, t + 1] = toks[i, 0]

        @pl.when(t < TMAX - 1)
        def _():
            _fetch_emb_and_project()


def _decode(eh0, eh1, ec1, wih, whh, bih, bhh, wq, wk, wv, wo, bq, bk, bv, bo,
            wlinT, blin, emb_tab):
    fixed = lambda t, v: (0, 0)
    vchunk = lambda t, v: (0, v)
    return pl.pallas_call(
        _dec_body,
        grid=(TMAX, NVC),
        in_specs=[
            pl.BlockSpec((B, H), fixed),
            pl.BlockSpec((B, H), fixed),
            pl.BlockSpec((B, H), fixed),
            pl.BlockSpec((NH4, EMB), fixed),
            pl.BlockSpec((NH4, H), fixed),
            pl.BlockSpec((1, NH4), fixed),
            pl.BlockSpec((1, NH4), fixed),
            pl.BlockSpec((H, H), fixed),
            pl.BlockSpec((H, H), fixed),
            pl.BlockSpec((H, H), fixed),
            pl.BlockSpec((H, H), fixed),
            pl.BlockSpec((1, H), fixed),
            pl.BlockSpec((1, H), fixed),
            pl.BlockSpec((1, H), fixed),
            pl.BlockSpec((1, H), fixed),
            pl.BlockSpec((CV, H), lambda t, v: (v, 0)),
            pl.BlockSpec((1, CV), vchunk),
            pl.BlockSpec(memory_space=pl.ANY),
        ],
        out_specs=[
            pl.BlockSpec((1, B, CV), lambda t, v: (t, 0, v)),
            pl.BlockSpec(memory_space=pltpu.MemorySpace.SMEM),
        ],
        out_shape=[
            jax.ShapeDtypeStruct((TMAX, B, V), jnp.float32),
            jax.ShapeDtypeStruct((B, TMAX + 1), jnp.int32),
        ],
        scratch_shapes=[
            pltpu.VMEM((B, H), jnp.float32),
            pltpu.VMEM((B, H), jnp.float32),
            pltpu.VMEM((B, H), jnp.float32),
            pltpu.VMEM((B, EMB), jnp.float32),
            pltpu.VMEM((B, H), jnp.float32),
            pltpu.VMEM((B, H), jnp.float32),
            pltpu.VMEM((B, H), jnp.float32),
            pltpu.VMEM((B, H), jnp.float32),
            pltpu.VMEM((B, 128), jnp.float32),
            pltpu.VMEM((B, 128), jnp.int32),
            pltpu.VMEM((B, NH4), jnp.float32),
            pltpu.VMEM((B, 1), jnp.int32),
            pltpu.SMEM((B, 1), jnp.int32),
            pltpu.SemaphoreType.DMA,
            pltpu.SemaphoreType.DMA,
        ],
        compiler_params=pltpu.CompilerParams(
            dimension_semantics=("arbitrary", "arbitrary"),
            vmem_limit_bytes=128 * 1024 * 1024,
        ),
    )(eh0, eh1, ec1, wih, whh, bih, bhh, wq, wk, wv, wo, bq, bk, bv, bo,
      wlinT, blin, emb_tab)


def kernel(params, text):
    p = params
    x = _embed_gather(p["enc_emb"], text.reshape(-1))
    h0f, h1f, c1f = _encoder_fused(
        x.reshape(S, B, EMB), p["enc_Wih0"], p["enc_Whh0"],
        p["enc_Wih1"], p["enc_Whh1"],
        p["enc_bih0"] + p["enc_bhh0"], p["enc_bih1"] + p["enc_bhh1"],
    )
    douts, res = _decode(
        h0f, h1f, c1f, p["dec_Wih"], p["dec_Whh"],
        p["dec_bih"].reshape(1, NH4), p["dec_bhh"].reshape(1, NH4),
        p["Wq"], p["Wk"], p["Wv"], p["Wo"],
        p["bq"].reshape(1, H), p["bk"].reshape(1, H),
        p["bv"].reshape(1, H), p["bo"].reshape(1, H),
        p["Wlin"], p["blin"].reshape(1, V), p["dec_emb"],
    )
    row0 = jnp.zeros((1, B, V), jnp.float32).at[:, :, 1].set(1.0)
    logits = jnp.transpose(jnp.concatenate([row0, douts], axis=0), (1, 2, 0))
    return res, logits


# speedup vs baseline: 1.5332x; 1.5332x over previous
import functools

import jax
import jax.numpy as jnp
from jax import lax
from jax.experimental import pallas as pl
from jax.experimental.pallas import tpu as pltpu
from jax.experimental.pallas import tpu_sc as plsc

V = 16000
EMB = 512
H = 768
NH4 = 4 * H
B = 8
S = 256
TMAX = 16
NVC = 5
CV = V // NVC


def _embed_gather(table, idx):
    info = plsc.get_sparse_core_info()
    nc, ns = info.num_cores, info.num_subcores
    nw = nc * ns
    n = idx.shape[0]
    d = table.shape[1]
    b_per_w = n // nw
    mesh = plsc.VectorSubcoreMesh(core_axis_name="c", subcore_axis_name="s")

    @functools.partial(
        pl.kernel,
        mesh=mesh,
        out_type=jax.ShapeDtypeStruct((n, d), jnp.float32),
        scratch_types=[
            pltpu.VMEM((b_per_w,), jnp.int32),
            pltpu.VMEM((b_per_w, d), jnp.float32),
            pltpu.SemaphoreType.DMA,
        ],
    )
    def k(table_hbm, idx_hbm, out_hbm, idx_v, rows_v, sem):
        wid = lax.axis_index("s") * nc + lax.axis_index("c")
        base = wid * b_per_w
        pltpu.sync_copy(idx_hbm.at[pl.ds(base, b_per_w)], idx_v)
        pltpu.async_copy(table_hbm.at[idx_v], rows_v, sem).wait()
        pltpu.sync_copy(rows_v, out_hbm.at[pl.ds(base, b_per_w)])

    return k(table, idx)


def _mm_body(x_ref, w_ref, b_ref, o_ref):
    o_ref[...] = (
        jnp.dot(x_ref[...], w_ref[...], preferred_element_type=jnp.float32)
        + b_ref[...]
    )


def _matmul_bias(x, wT, b):
    m, k = x.shape
    n = wT.shape[1]
    bn = 768
    return pl.pallas_call(
        _mm_body,
        grid=(n // bn,),
        in_specs=[
            pl.BlockSpec((m, k), lambda j: (0, 0)),
            pl.BlockSpec((k, bn), lambda j: (0, j)),
            pl.BlockSpec((1, bn), lambda j: (0, j)),
        ],
        out_specs=pl.BlockSpec((m, bn), lambda j: (0, j)),
        out_shape=jax.ShapeDtypeStruct((m, n), jnp.float32),
        compiler_params=pltpu.CompilerParams(
            dimension_semantics=("arbitrary",),
        ),
    )(x, wT, b.reshape(1, n))


def _lstm_body(g_ref, whh_ref, hseq_ref, cfin_ref, h_sc, c_sc):
    t = pl.program_id(0)

    @pl.when(t == 0)
    def _():
        h_sc[...] = jnp.zeros_like(h_sc)
        c_sc[...] = jnp.zeros_like(c_sc)

    g = g_ref[0] + jnp.dot(
        h_sc[...], whh_ref[...], preferred_element_type=jnp.float32
    )
    ig = jax.nn.sigmoid(g[:, :H])
    fg = jax.nn.sigmoid(g[:, H : 2 * H])
    gg = jnp.tanh(g[:, 2 * H : 3 * H])
    og = jax.nn.sigmoid(g[:, 3 * H :])
    c2 = fg * c_sc[...] + ig * gg
    h2 = og * jnp.tanh(c2)
    h_sc[...] = h2
    c_sc[...] = c2
    hseq_ref[0] = h2

    @pl.when(t == S - 1)
    def _():
        cfin_ref[...] = c2


def _mm8_body(x_ref, w_ref, b_ref, o_ref):
    o_ref[0] = (
        jnp.dot(x_ref[0], w_ref[...], preferred_element_type=jnp.float32)
        + b_ref[...]
    )


def _matmul_rows(xseq, wT, b):
    k = xseq.shape[2]
    return pl.pallas_call(
        _mm8_body,
        grid=(S,),
        in_specs=[
            pl.BlockSpec((1, B, k), lambda t: (t, 0, 0)),
            pl.BlockSpec((k, NH4), lambda t: (0, 0)),
            pl.BlockSpec((1, NH4), lambda t: (0, 0)),
        ],
        out_specs=pl.BlockSpec((1, B, NH4), lambda t: (t, 0, 0)),
        out_shape=jax.ShapeDtypeStruct((S, B, NH4), jnp.float32),
        compiler_params=pltpu.CompilerParams(
            dimension_semantics=("arbitrary",),
        ),
    )(xseq, wT, b.reshape(1, NH4))


def _lstm_scan(g, whhT):
    return pl.pallas_call(
        _lstm_body,
        grid=(S,),
        in_specs=[
            pl.BlockSpec((1, B, NH4), lambda t: (t, 0, 0)),
            pl.BlockSpec((H, NH4), lambda t: (0, 0)),
        ],
        out_specs=[
            pl.BlockSpec((1, B, H), lambda t: (t, 0, 0)),
            pl.BlockSpec((B, H), lambda t: (0, 0)),
        ],
        out_shape=[
            jax.ShapeDtypeStruct((S, B, H), jnp.float32),
            jax.ShapeDtypeStruct((B, H), jnp.float32),
        ],
        scratch_shapes=[
            pltpu.VMEM((B, H), jnp.float32),
            pltpu.VMEM((B, H), jnp.float32),
        ],
        compiler_params=pltpu.CompilerParams(
            dimension_semantics=("arbitrary",),
        ),
    )(g, whhT)


def _dec_body(
    eh0, eh1, ec1, wih, whh, bih, bhh, wq, wk, wv, wo, bq, bk, bv, bo,
    wlin, blin, emb_tab,
    douts_ref, res_ref,
    h_sc, c_sc, attn_sc, emb_sc, ke0, ke1, ve0, ve1, cv_sc, ct_sc,
    g_sc, tokv, toks, esem, psem,
):
    t = pl.program_id(0)
    v = pl.program_id(1)

    def _fetch_emb_and_project():
        for i in range(B):
            pltpu.make_async_copy(
                emb_tab.at[pl.ds(toks[i, 0], 1), :],
                emb_sc.at[pl.ds(i, 1), :],
                esem,
            ).start()
        for i in range(B):
            pltpu.make_async_copy(
                emb_tab.at[pl.ds(toks[i, 0], 1), :],
                emb_sc.at[pl.ds(i, 1), :],
                esem,
            ).wait()
        g_sc[...] = jnp.dot(
            emb_sc[...], wih[...], preferred_element_type=jnp.float32
        )

    @pl.when(v == 0)
    def _step_head():
        @pl.when(t == 0)
        def _():
            h_sc[...] = eh1[...]
            c_sc[...] = ec1[...]
            ke0[...] = jnp.dot(eh0[...], wk[...], preferred_element_type=jnp.float32) + bk[...]
            ke1[...] = jnp.dot(eh1[...], wk[...], preferred_element_type=jnp.float32) + bk[...]
            ve0[...] = jnp.dot(eh0[...], wv[...], preferred_element_type=jnp.float32) + bv[...]
            ve1[...] = jnp.dot(eh1[...], wv[...], preferred_element_type=jnp.float32) + bv[...]
            for i in range(B):
                toks[i, 0] = 1
            _fetch_emb_and_project()

        cv_sc[...] = jnp.full_like(cv_sc, -jnp.inf)

        g = (
            g_sc[...]
            + jnp.dot(h_sc[...], whh[...], preferred_element_type=jnp.float32)
            + bih[...]
            + bhh[...]
        )
        ig = jax.nn.sigmoid(g[:, :H])
        fg = jax.nn.sigmoid(g[:, H : 2 * H])
        gg = jnp.tanh(g[:, 2 * H : 3 * H])
        og = jax.nn.sigmoid(g[:, 3 * H :])
        c2 = fg * c_sc[...] + ig * gg
        h2 = og * jnp.tanh(c2)
        h_sc[...] = h2
        c_sc[...] = c2

        q = jnp.dot(h2, wq[...], preferred_element_type=jnp.float32) + bq[...]
        kd = jnp.dot(h2, wk[...], preferred_element_type=jnp.float32) + bk[...]
        vd = jnp.dot(h2, wv[...], preferred_element_type=jnp.float32) + bv[...]
        kall = jnp.concatenate([ke0[...], ke1[...], kd], axis=0)
        vall = jnp.concatenate([ve0[...], ve1[...], vd], axis=0)
        s24 = lax.dot_general(
            q, kall, (((1,), (1,)), ((), ())),
            preferred_element_type=jnp.float32,
        )
        lane24 = lax.broadcasted_iota(jnp.int32, (B, 3 * B), 1)
        row24 = lax.broadcasted_iota(jnp.int32, (B, 3 * B), 0)
        sq = jnp.sqrt(jnp.float32(H))
        sel = [lane24 == (j * B + row24) for j in range(3)]
        s0 = jnp.sum(jnp.where(sel[0], s24, 0.0), 1, keepdims=True) / sq
        s1 = jnp.sum(jnp.where(sel[1], s24, 0.0), 1, keepdims=True) / sq
        s2 = jnp.sum(jnp.where(sel[2], s24, 0.0), 1, keepdims=True) / sq
        mx = jnp.maximum(jnp.maximum(s0, s1), s2)
        e0 = jnp.exp(s0 - mx)
        e1 = jnp.exp(s1 - mx)
        e2 = jnp.exp(s2 - mx)
        den = e0 + e1 + e2
        wfull = (
            jnp.where(sel[0], e0 / den, 0.0)
            + jnp.where(sel[1], e1 / den, 0.0)
            + jnp.where(sel[2], e2 / den, 0.0)
        )
        at = jnp.dot(wfull, vall, preferred_element_type=jnp.float32)
        attn_sc[...] = jnp.dot(at, wo[...], preferred_element_type=jnp.float32) + bo[...]

    logits = (
        jnp.dot(attn_sc[...], wlin[...], preferred_element_type=jnp.float32)
        + blin[...]
    )
    douts_ref[0] = logits
    lane = lax.broadcasted_iota(jnp.int32, (B, CV), 1)
    mx = jnp.max(logits, axis=1, keepdims=True)
    l0 = jnp.min(jnp.where(logits == mx, lane, CV), axis=1, keepdims=True)
    lane128 = lax.broadcasted_iota(jnp.int32, (B, 128), 1)
    upd = lane128 == v
    cv_sc[...] = jnp.where(upd, mx, cv_sc[...])
    ct_sc[...] = jnp.where(upd, l0 + v * CV, ct_sc[...])

    @pl.when(v == NVC - 1)
    def _step_tail():
        mz = jnp.max(cv_sc[...], axis=1, keepdims=True)
        lz = jnp.min(
            jnp.where(cv_sc[...] == mz, lane128, 128), axis=1, keepdims=True
        )
        tokv[...] = jnp.sum(
            jnp.where(lane128 == lz, ct_sc[...], 0), axis=1, keepdims=True
        )
        cp = pltpu.make_async_copy(tokv, toks, psem)
        cp.start()
        cp.wait()

        @pl.when(t == 0)
        def _():
            for i in range(B):
                res_ref[i, 0] = 1

        for i in range(B):
            res_ref[i, t + 1] = toks[i, 0]

        @pl.when(t < TMAX - 1)
        def _():
            _fetch_emb_and_project()


def _decode(eh0, eh1, ec1, wih, whh, bih, bhh, wq, wk, wv, wo, bq, bk, bv, bo,
            wlinT, blin, emb_tab):
    fixed = lambda t, v: (0, 0)
    vchunk = lambda t, v: (0, v)
    return pl.pallas_call(
        _dec_body,
        grid=(TMAX, NVC),
        in_specs=[
            pl.BlockSpec((B, H), fixed),
            pl.BlockSpec((B, H), fixed),
            pl.BlockSpec((B, H), fixed),
            pl.BlockSpec((EMB, NH4), fixed),
            pl.BlockSpec((H, NH4), fixed),
            pl.BlockSpec((1, NH4), fixed),
            pl.BlockSpec((1, NH4), fixed),
            pl.BlockSpec((H, H), fixed),
            pl.BlockSpec((H, H), fixed),
            pl.BlockSpec((H, H), fixed),
            pl.BlockSpec((H, H), fixed),
            pl.BlockSpec((1, H), fixed),
            pl.BlockSpec((1, H), fixed),
            pl.BlockSpec((1, H), fixed),
            pl.BlockSpec((1, H), fixed),
            pl.BlockSpec((H, CV), vchunk),
            pl.BlockSpec((1, CV), vchunk),
            pl.BlockSpec(memory_space=pl.ANY),
        ],
        out_specs=[
            pl.BlockSpec((1, B, CV), lambda t, v: (t, 0, v)),
            pl.BlockSpec(memory_space=pltpu.MemorySpace.SMEM),
        ],
        out_shape=[
            jax.ShapeDtypeStruct((TMAX, B, V), jnp.float32),
            jax.ShapeDtypeStruct((B, TMAX + 1), jnp.int32),
        ],
        scratch_shapes=[
            pltpu.VMEM((B, H), jnp.float32),
            pltpu.VMEM((B, H), jnp.float32),
            pltpu.VMEM((B, H), jnp.float32),
            pltpu.VMEM((B, EMB), jnp.float32),
            pltpu.VMEM((B, H), jnp.float32),
            pltpu.VMEM((B, H), jnp.float32),
            pltpu.VMEM((B, H), jnp.float32),
            pltpu.VMEM((B, H), jnp.float32),
            pltpu.VMEM((B, 128), jnp.float32),
            pltpu.VMEM((B, 128), jnp.int32),
            pltpu.VMEM((B, NH4), jnp.float32),
            pltpu.VMEM((B, 1), jnp.int32),
            pltpu.SMEM((B, 1), jnp.int32),
            pltpu.SemaphoreType.DMA,
            pltpu.SemaphoreType.DMA,
        ],
        compiler_params=pltpu.CompilerParams(
            dimension_semantics=("arbitrary", "arbitrary"),
            vmem_limit_bytes=128 * 1024 * 1024,
        ),
    )(eh0, eh1, ec1, wih, whh, bih, bhh, wq, wk, wv, wo, bq, bk, bv, bo,
      wlinT, blin, emb_tab)


def kernel(params, text):
    p = params
    x = _embed_gather(p["enc_emb"], text.reshape(-1))
    g0 = _matmul_bias(x, p["enc_Wih0"].T, p["enc_bih0"] + p["enc_bhh0"])
    h0seq, _ = _lstm_scan(g0.reshape(S, B, NH4), p["enc_Whh0"].T)
    g1 = _matmul_rows(h0seq, p["enc_Wih1"].T, p["enc_bih1"] + p["enc_bhh1"])
    h1seq, c1 = _lstm_scan(g1, p["enc_Whh1"].T)
    douts, res = _decode(
        h0seq[-1], h1seq[-1], c1, p["dec_Wih"].T, p["dec_Whh"].T,
        p["dec_bih"].reshape(1, NH4), p["dec_bhh"].reshape(1, NH4),
        p["Wq"].T, p["Wk"].T, p["Wv"].T, p["Wo"].T,
        p["bq"].reshape(1, H), p["bk"].reshape(1, H),
        p["bv"].reshape(1, H), p["bo"].reshape(1, H),
        p["Wlin"].T, p["blin"].reshape(1, V), p["dec_emb"],
    )
    row0 = jnp.zeros((1, B, V), jnp.float32).at[:, :, 1].set(1.0)
    logits = jnp.transpose(jnp.concatenate([row0, douts], axis=0), (1, 2, 0))
    return res, logits
